# baseline (speedup 1.0000x reference)
import functools

import jax
import jax.numpy as jnp
import numpy as np
from jax import lax
from jax.experimental import pallas as pl
from jax.experimental.pallas import tpu as pltpu
from jax.experimental.pallas import tpu_sc as plsc

_EDGE_SRC = np.array([0, 1, 1, 1, 1, 2, 2, 2, 3, 3, 3, 4, 4, 4, 5, 5, 5, 5, 6, 6, 6, 6, 6, 7, 7, 7, 8, 8, 8, 9, 9, 9, 10, 10, 10, 11, 11, 11, 12, 12, 12, 13, 13, 13, 14, 14, 14, 15, 15, 15, 16, 16, 16, 17, 17, 17, 18, 18, 18, 18, 19, 20, 21, 21, 21, 22, 23, 23, 23, 24, 25, 26], dtype=np.int32)
_EDGE_DST = np.array([1, 0, 2, 4, 6, 1, 3, 7, 2, 4, 24, 1, 3, 5, 4, 6, 17, 19, 1, 5, 7, 8, 10, 2, 6, 23, 6, 9, 16, 8, 10, 13, 6, 9, 11, 10, 12, 23, 11, 13, 21, 9, 12, 14, 13, 15, 20, 14, 16, 18, 8, 15, 17, 5, 16, 18, 15, 17, 25, 26, 5, 14, 12, 22, 23, 21, 7, 11, 21, 3, 18, 18], dtype=np.int32)
_N = 72
_B, _C, _F = 32, 128, 4096
_NCHUNK, _CW = 10, 71
_P = _NCHUNK * _CW
_PPAD = 720
_GCHUNK = 120

_A_np = np.zeros((_N, _N), dtype=np.float32)
for _s, _d in zip(_EDGE_SRC, _EDGE_DST):
    _A_np[_d, _s] += 1.0
_A0_np = np.concatenate([_A_np[:, 1:], _A_np[:, :1]], axis=1).copy()
_A2_np = np.concatenate([_A_np[1:, :], _A_np[:1, :]], axis=0).copy()
_S_np = np.zeros((_B, _B * _N), dtype=np.float32)
for _i in range(_B):
    _S_np[_i, _i * _N + _N - 1] = 1.0

_rng = np.random.default_rng(0)
_II = np.stack([_rng.choice(_F, _P, replace=False) for _ in range(_B)])
_IDX_np = np.zeros((_B, _PPAD), dtype=np.int32)
for _i in range(_B):
    _IDX_np[_i, :_P] = _II[_i].astype(np.int32) + _i * _F

_MMARGS = dict(dimension_numbers=(((1,), (0,)), ((), ())),
               preferred_element_type=jnp.float32)


def _mm(a, b):
    return lax.dot_general(a, b, **_MMARGS)


def _mmh(a, b):
    return lax.dot_general(a, b, precision=lax.Precision.HIGHEST, **_MMARGS)


def _k1_body(x_ref, xt_ref):
    xt_ref[0] = x_ref[0].T


def _transpose_call(x):
    fchunk = 1024
    return pl.pallas_call(
        _k1_body,
        grid=(_B, _F // fchunk),
        in_specs=[pl.BlockSpec((1, _C, fchunk), lambda s, k: (s, 0, k))],
        out_specs=pl.BlockSpec((1, fchunk, _C), lambda s, k: (s, k, 0)),
        out_shape=jax.ShapeDtypeStruct((_B, _F, _C), jnp.float32),
    )(x)


def _k2_body(xt_hbm, idx_hbm, xg_hbm, idx_v, rows_v, sem):
    wid = lax.axis_index("s") * 2 + lax.axis_index("c")
    pltpu.sync_copy(idx_hbm.at[wid], idx_v)
    for c in range(_PPAD // _GCHUNK):
        pltpu.async_copy(
            xt_hbm.at[idx_v.at[pl.ds(c * _GCHUNK, _GCHUNK)]],
            rows_v.at[pl.ds(c * _GCHUNK, _GCHUNK)], sem).wait()
    for j in range(_NCHUNK):
        pltpu.sync_copy(rows_v.at[pl.ds(j * _CW, _N)],
                        xg_hbm.at[j, pl.ds(wid * _N, _N)])


def _gather_call(xt, idx):
    mesh = plsc.VectorSubcoreMesh(core_axis_name="c", subcore_axis_name="s")
    k = pl.kernel(
        _k2_body,
        out_type=jax.ShapeDtypeStruct((_NCHUNK, _B * _N, _C), jnp.float32),
        mesh=mesh,
        scratch_types=[
            pltpu.VMEM((_PPAD,), jnp.int32),
            pltpu.VMEM((_PPAD, _C), jnp.float32),
            pltpu.SemaphoreType.DMA,
        ],
    )
    return k(xt.reshape(_B * _F, _C), idx)


def _blockagg(M, h):
    return jnp.concatenate(
        [_mmh(M, h[s * _N:(s + 1) * _N]) for s in range(_B)], axis=0)


def _k3_body(xg_ref, A_ref, A0_ref, A2_ref, S_ref, W0_ref, b0_ref, W1_ref,
             b1_ref, W2_ref, b2_ref, Wc_ref, bc_ref, out_ref, X3_ref):
    j = pl.program_id(0)

    @pl.when(j == 0)
    def _():
        X3_ref[...] = jnp.zeros((_B * _N, _C), jnp.float32)

    mask71 = lax.rem(
        lax.broadcasted_iota(jnp.int32, (_B * _N, 1), 0), _N) == (_N - 1)
    Xin = jnp.where(mask71, X3_ref[...], xg_ref[0])
    X = jnp.maximum(_blockagg(A0_ref[...], _mm(Xin, W0_ref[...])) + b0_ref[...], 0.0)
    X = jnp.maximum(_blockagg(A_ref[...], _mm(X, W1_ref[...])) + b1_ref[...], 0.0)
    X = jnp.maximum(_blockagg(A2_ref[...], _mm(X, W2_ref[...])) + b2_ref[...], 0.0)
    X3_ref[...] = X

    @pl.when(j == _NCHUNK - 1)
    def _():
        fb = _mmh(S_ref[...], X)
        out_ref[...] = _mm(fb, Wc_ref[...]) + bc_ref[...]


def _gcn_call(xg, A, A0, A2, S, W0, b0, W1, b1, W2, b2, Wc, bc):
    full = lambda *shape: pl.BlockSpec(shape, lambda j: (0,) * len(shape))
    return pl.pallas_call(
        _k3_body,
        grid=(_NCHUNK,),
        in_specs=[
            pl.BlockSpec((1, _B * _N, _C), lambda j: (j, 0, 0)),
            full(_N, _N), full(_N, _N), full(_N, _N),
            full(_B, _B * _N),
            full(_C, 256), full(1, 256),
            full(256, 256), full(1, 256),
            full(256, _C), full(1, _C),
            full(_C, 1), full(1, 1),
        ],
        out_specs=pl.BlockSpec((_B, 1), lambda j: (0, 0)),
        out_shape=jax.ShapeDtypeStruct((_B, 1), jnp.float32),
        scratch_shapes=[pltpu.VMEM((_B * _N, _C), jnp.float32)],
    )(xg, A, A0, A2, S, W0, b0, W1, b1, W2, b2, Wc, bc)


def kernel(x, W0, b0, W1, b1, W2, b2, Wc, bc):
    xt = _transpose_call(x)
    xg = _gather_call(xt, jnp.asarray(_IDX_np))
    return _gcn_call(
        xg, jnp.asarray(_A_np), jnp.asarray(_A0_np), jnp.asarray(_A2_np),
        jnp.asarray(_S_np),
        W0, b0.reshape(1, -1), W1, b1.reshape(1, -1), W2, b2.reshape(1, -1),
        Wc, bc.reshape(1, 1))


# speedup vs baseline: 1.4146x; 1.4146x over previous
import functools

import jax
import jax.numpy as jnp
import numpy as np
from jax import lax
from jax.experimental import pallas as pl
from jax.experimental.pallas import tpu as pltpu
from jax.experimental.pallas import tpu_sc as plsc

_EDGE_SRC = np.array([0, 1, 1, 1, 1, 2, 2, 2, 3, 3, 3, 4, 4, 4, 5, 5, 5, 5, 6, 6, 6, 6, 6, 7, 7, 7, 8, 8, 8, 9, 9, 9, 10, 10, 10, 11, 11, 11, 12, 12, 12, 13, 13, 13, 14, 14, 14, 15, 15, 15, 16, 16, 16, 17, 17, 17, 18, 18, 18, 18, 19, 20, 21, 21, 21, 22, 23, 23, 23, 24, 25, 26], dtype=np.int32)
_EDGE_DST = np.array([1, 0, 2, 4, 6, 1, 3, 7, 2, 4, 24, 1, 3, 5, 4, 6, 17, 19, 1, 5, 7, 8, 10, 2, 6, 23, 6, 9, 16, 8, 10, 13, 6, 9, 11, 10, 12, 23, 11, 13, 21, 9, 12, 14, 13, 15, 20, 14, 16, 18, 8, 15, 17, 5, 16, 18, 15, 17, 25, 26, 5, 14, 12, 22, 23, 21, 7, 11, 21, 3, 18, 18], dtype=np.int32)
_N = 72
_B, _C, _F = 32, 128, 4096
_NCHUNK, _CW = 10, 71
_NE = 27
_NP = 32
_CD = _NE - 1
_PPAD = 272
_GCH = (128, 128, 16)

_A_np = np.zeros((_NP, _NP), dtype=np.float32)
for _s, _d in zip(_EDGE_SRC, _EDGE_DST):
    _A_np[_d, _s] += 1.0
_A0_np = np.zeros((_NP, _NP), dtype=np.float32)
_A0_np[:, :_CD] = _A_np[:, 1:_NE]
_A0_np[:, _NP - 1] = _A_np[:, 0]
_A2_np = np.zeros((_NP, _NP), dtype=np.float32)
_A2_np[:_CD, :] = _A_np[1:_NE, :]
_A2_np[_NP - 1, :] = _A_np[0, :]
_S_np = np.zeros((_B, _B * _NP), dtype=np.float32)
for _i in range(_B):
    _S_np[_i, _i * _NP + _NP - 1] = 1.0

_rng = np.random.default_rng(0)
_II = np.stack([_rng.choice(_F, _NCHUNK * _CW, replace=False)
                for _ in range(_B)])
_IDX_np = np.zeros((_B, _PPAD), dtype=np.int32)
for _i in range(_B):
    for _j in range(_NCHUNK):
        _IDX_np[_i, _j * _CD:(_j + 1) * _CD] = (
            _II[_i, _j * _CW:_j * _CW + _CD].astype(np.int32) + _i * _F)
    _IDX_np[_i, _NCHUNK * _CD:] = _i * _F

_MMARGS = dict(dimension_numbers=(((1,), (0,)), ((), ())),
               preferred_element_type=jnp.float32)


def _mm(a, b):
    return lax.dot_general(a, b, **_MMARGS)


def _mmh(a, b):
    return lax.dot_general(a, b, precision=lax.Precision.HIGHEST, **_MMARGS)


def _k1_body(x_ref, xt_ref):
    xt_ref[0] = x_ref[0].T


def _transpose_call(x):
    fchunk = 1024
    return pl.pallas_call(
        _k1_body,
        grid=(_B, _F // fchunk),
        in_specs=[pl.BlockSpec((1, _C, fchunk), lambda s, k: (s, 0, k))],
        out_specs=pl.BlockSpec((1, fchunk, _C), lambda s, k: (s, k, 0)),
        out_shape=jax.ShapeDtypeStruct((_B, _F, _C), jnp.float32),
    )(x)


def _k2_body(xt_hbm, idx_hbm, xg_hbm, idx_v, rows_v, sem):
    wid = lax.axis_index("s") * 2 + lax.axis_index("c")
    pltpu.sync_copy(idx_hbm.at[wid], idx_v)
    off = 0
    copies = []
    for g in _GCH:
        copies.append(pltpu.async_copy(
            xt_hbm.at[idx_v.at[pl.ds(off, g)]],
            rows_v.at[pl.ds(off, g)], sem))
        off += g
    for c in copies:
        c.wait()
    for j in range(_NCHUNK):
        pltpu.sync_copy(rows_v.at[pl.ds(j * _CD, _NP)],
                        xg_hbm.at[j, pl.ds(wid * _NP, _NP)])


def _gather_call(xt, idx):
    mesh = plsc.VectorSubcoreMesh(core_axis_name="c", subcore_axis_name="s")
    k = pl.kernel(
        _k2_body,
        out_type=jax.ShapeDtypeStruct((_NCHUNK, _B * _NP, _C), jnp.float32),
        mesh=mesh,
        scratch_types=[
            pltpu.VMEM((_PPAD,), jnp.int32),
            pltpu.VMEM((_PPAD, _C), jnp.float32),
            pltpu.SemaphoreType.DMA,
        ],
    )
    return k(xt.reshape(_B * _F, _C), idx)


def _blockagg(M, h):
    return jnp.concatenate(
        [_mmh(M, h[s * _NP:(s + 1) * _NP]) for s in range(_B)], axis=0)


def _k3_body(xg_ref, A_ref, A0_ref, A2_ref, S_ref, W0_ref, b0_ref, W1_ref,
             b1_ref, W2_ref, b2_ref, Wc_ref, bc_ref, out_ref, X3_ref):
    j = pl.program_id(0)

    @pl.when(j == 0)
    def _():
        X3_ref[...] = jnp.zeros((_B * _NP, _C), jnp.float32)

    maskfb = lax.rem(
        lax.broadcasted_iota(jnp.int32, (_B * _NP, 1), 0), _NP) == (_NP - 1)
    Xin = jnp.where(maskfb, X3_ref[...], xg_ref[0])
    X = jnp.maximum(_blockagg(A0_ref[...], _mm(Xin, W0_ref[...])) + b0_ref[...], 0.0)
    X = jnp.maximum(_blockagg(A_ref[...], _mm(X, W1_ref[...])) + b1_ref[...], 0.0)
    X = jnp.maximum(_blockagg(A2_ref[...], _mm(X, W2_ref[...])) + b2_ref[...], 0.0)
    X3_ref[...] = X

    @pl.when(j == _NCHUNK - 1)
    def _():
        fb = _mmh(S_ref[...], X)
        out_ref[...] = _mm(fb, Wc_ref[...]) + bc_ref[...]


def _gcn_call(xg, A, A0, A2, S, W0, b0, W1, b1, W2, b2, Wc, bc):
    full = lambda *shape: pl.BlockSpec(shape, lambda j: (0,) * len(shape))
    return pl.pallas_call(
        _k3_body,
        grid=(_NCHUNK,),
        in_specs=[
            pl.BlockSpec((1, _B * _NP, _C), lambda j: (j, 0, 0)),
            full(_NP, _NP), full(_NP, _NP), full(_NP, _NP),
            full(_B, _B * _NP),
            full(_C, 256), full(1, 256),
            full(256, 256), full(1, 256),
            full(256, _C), full(1, _C),
            full(_C, 1), full(1, 1),
        ],
        out_specs=pl.BlockSpec((_B, 1), lambda j: (0, 0)),
        out_shape=jax.ShapeDtypeStruct((_B, 1), jnp.float32),
        scratch_shapes=[pltpu.VMEM((_B * _NP, _C), jnp.float32)],
    )(xg, A, A0, A2, S, W0, b0, W1, b1, W2, b2, Wc, bc)


def kernel(x, W0, b0, W1, b1, W2, b2, Wc, bc):
    xt = _transpose_call(x)
    xg = _gather_call(xt, jnp.asarray(_IDX_np))
    return _gcn_call(
        xg, jnp.asarray(_A_np), jnp.asarray(_A0_np), jnp.asarray(_A2_np),
        jnp.asarray(_S_np),
        W0, b0.reshape(1, -1), W1, b1.reshape(1, -1), W2, b2.reshape(1, -1),
        Wc, bc.reshape(1, 1))


# speedup vs baseline: 2.0451x; 1.4457x over previous
import functools

import jax
import jax.numpy as jnp
import numpy as np
from jax import lax
from jax.experimental import pallas as pl
from jax.experimental.pallas import tpu as pltpu
from jax.experimental.pallas import tpu_sc as plsc

_EDGE_SRC = np.array([0, 1, 1, 1, 1, 2, 2, 2, 3, 3, 3, 4, 4, 4, 5, 5, 5, 5, 6, 6, 6, 6, 6, 7, 7, 7, 8, 8, 8, 9, 9, 9, 10, 10, 10, 11, 11, 11, 12, 12, 12, 13, 13, 13, 14, 14, 14, 15, 15, 15, 16, 16, 16, 17, 17, 17, 18, 18, 18, 18, 19, 20, 21, 21, 21, 22, 23, 23, 23, 24, 25, 26], dtype=np.int32)
_EDGE_DST = np.array([1, 0, 2, 4, 6, 1, 3, 7, 2, 4, 24, 1, 3, 5, 4, 6, 17, 19, 1, 5, 7, 8, 10, 2, 6, 23, 6, 9, 16, 8, 10, 13, 6, 9, 11, 10, 12, 23, 11, 13, 21, 9, 12, 14, 13, 15, 20, 14, 16, 18, 8, 15, 17, 5, 16, 18, 15, 17, 25, 26, 5, 14, 12, 22, 23, 21, 7, 11, 21, 3, 18, 18], dtype=np.int32)
_N = 72
_B, _C, _F = 32, 128, 4096
_NCHUNK, _CW = 10, 71
_NE = 27
_NP = 32
_CD = _NE - 1
_PPAD = 272
_GCH = (128, 128, 16)

_A_np = np.zeros((_NP, _NP), dtype=np.float32)
for _s, _d in zip(_EDGE_SRC, _EDGE_DST):
    _A_np[_d, _s] += 1.0
_A0_np = np.zeros((_NP, _NP), dtype=np.float32)
_A0_np[:, :_CD] = _A_np[:, 1:_NE]
_A0_np[:, _NP - 1] = _A_np[:, 0]
_A2_np = np.zeros((_NP, _NP), dtype=np.float32)
_A2_np[:_CD, :] = _A_np[1:_NE, :]
_A2_np[_NP - 1, :] = _A_np[0, :]
_S_np = np.zeros((_B, _B * _NP), dtype=np.float32)
for _i in range(_B):
    _S_np[_i, _i * _NP + _NP - 1] = 1.0

_rng = np.random.default_rng(0)
_II = np.stack([_rng.choice(_F, _NCHUNK * _CW, replace=False)
                for _ in range(_B)])
_IDX_np = np.zeros((_B, _PPAD), dtype=np.int32)
for _i in range(_B):
    for _j in range(_NCHUNK):
        _IDX_np[_i, _j * _CD:(_j + 1) * _CD] = (
            _II[_i, _j * _CW:_j * _CW + _CD].astype(np.int32) + _i * _F)
    _IDX_np[_i, _NCHUNK * _CD:] = _i * _F

_MMARGS = dict(dimension_numbers=(((1,), (0,)), ((), ())),
               preferred_element_type=jnp.float32)


def _mm(a, b):
    return lax.dot_general(a, b, **_MMARGS)


def _mmh(a, b):
    return lax.dot_general(a, b, precision=lax.Precision.HIGHEST, **_MMARGS)


def _k1_body(x_ref, xt_ref):
    xt_ref[0] = x_ref[0].T


def _transpose_call(x):
    return pl.pallas_call(
        _k1_body,
        grid=(_B,),
        in_specs=[pl.BlockSpec((1, _C, _F), lambda s: (s, 0, 0))],
        out_specs=pl.BlockSpec((1, _F, _C), lambda s: (s, 0, 0)),
        out_shape=jax.ShapeDtypeStruct((_B, _F, _C), jnp.float32),
    )(x)


def _k2_body(xt_hbm, idx_hbm, xg_hbm, idx_v, rows_v, sem):
    wid = lax.axis_index("s") * 2 + lax.axis_index("c")
    pltpu.sync_copy(idx_hbm.at[wid], idx_v)
    off = 0
    copies = []
    for g in _GCH:
        copies.append(pltpu.async_copy(
            xt_hbm.at[idx_v.at[pl.ds(off, g)]],
            rows_v.at[pl.ds(off, g)], sem))
        off += g
    for c in copies:
        c.wait()
    for j in range(_NCHUNK):
        pltpu.sync_copy(rows_v.at[pl.ds(j * _CD, _NP)],
                        xg_hbm.at[j, pl.ds(wid * _NP, _NP)])


def _gather_call(xt, idx):
    mesh = plsc.VectorSubcoreMesh(core_axis_name="c", subcore_axis_name="s")
    k = pl.kernel(
        _k2_body,
        out_type=jax.ShapeDtypeStruct((_NCHUNK, _B * _NP, _C), jnp.float32),
        mesh=mesh,
        scratch_types=[
            pltpu.VMEM((_PPAD,), jnp.int32),
            pltpu.VMEM((_PPAD, _C), jnp.float32),
            pltpu.SemaphoreType.DMA,
        ],
    )
    return k(xt.reshape(_B * _F, _C), idx)


def _blockagg(M, h):
    return jnp.concatenate(
        [_mmh(M, h[s * _NP:(s + 1) * _NP]) for s in range(_B)], axis=0)


def _k3_body(xg_ref, A_ref, A0_ref, A2_ref, S_ref, W0_ref, b0_ref, W1_ref,
             b1_ref, W2_ref, b2_ref, Wc_ref, bc_ref, out_ref, X3_ref):
    j = pl.program_id(0)

    @pl.when(j == 0)
    def _():
        X3_ref[...] = jnp.zeros((_B * _NP, _C), jnp.float32)

    maskfb = lax.rem(
        lax.broadcasted_iota(jnp.int32, (_B * _NP, 1), 0), _NP) == (_NP - 1)
    Xin = jnp.where(maskfb, X3_ref[...], xg_ref[0])
    X = jnp.maximum(_blockagg(A0_ref[...], _mm(Xin, W0_ref[...])) + b0_ref[...], 0.0)
    X = jnp.maximum(_blockagg(A_ref[...], _mm(X, W1_ref[...])) + b1_ref[...], 0.0)
    X = jnp.maximum(_blockagg(A2_ref[...], _mm(X, W2_ref[...])) + b2_ref[...], 0.0)
    X3_ref[...] = X

    @pl.when(j == _NCHUNK - 1)
    def _():
        fb = _mmh(S_ref[...], X)
        out_ref[...] = _mm(fb, Wc_ref[...]) + bc_ref[...]


def _gcn_call(xg, A, A0, A2, S, W0, b0, W1, b1, W2, b2, Wc, bc):
    full = lambda *shape: pl.BlockSpec(shape, lambda j: (0,) * len(shape))
    return pl.pallas_call(
        _k3_body,
        grid=(_NCHUNK,),
        in_specs=[
            pl.BlockSpec((1, _B * _NP, _C), lambda j: (j, 0, 0)),
            full(_NP, _NP), full(_NP, _NP), full(_NP, _NP),
            full(_B, _B * _NP),
            full(_C, 256), full(1, 256),
            full(256, 256), full(1, 256),
            full(256, _C), full(1, _C),
            full(_C, 1), full(1, 1),
        ],
        out_specs=pl.BlockSpec((_B, 1), lambda j: (0, 0)),
        out_shape=jax.ShapeDtypeStruct((_B, 1), jnp.float32),
        scratch_shapes=[pltpu.VMEM((_B * _NP, _C), jnp.float32)],
    )(xg, A, A0, A2, S, W0, b0, W1, b1, W2, b2, Wc, bc)


def kernel(x, W0, b0, W1, b1, W2, b2, Wc, bc):
    xt = _transpose_call(x)
    xg = _gather_call(xt, jnp.asarray(_IDX_np))
    return _gcn_call(
        xg, jnp.asarray(_A_np), jnp.asarray(_A0_np), jnp.asarray(_A2_np),
        jnp.asarray(_S_np),
        W0, b0.reshape(1, -1), W1, b1.reshape(1, -1), W2, b2.reshape(1, -1),
        Wc, bc.reshape(1, 1))


# speedup vs baseline: 2.0494x; 1.0021x over previous
import functools

import jax
import jax.numpy as jnp
import numpy as np
from jax import lax
from jax.experimental import pallas as pl
from jax.experimental.pallas import tpu as pltpu
from jax.experimental.pallas import tpu_sc as plsc

_EDGE_SRC = np.array([0, 1, 1, 1, 1, 2, 2, 2, 3, 3, 3, 4, 4, 4, 5, 5, 5, 5, 6, 6, 6, 6, 6, 7, 7, 7, 8, 8, 8, 9, 9, 9, 10, 10, 10, 11, 11, 11, 12, 12, 12, 13, 13, 13, 14, 14, 14, 15, 15, 15, 16, 16, 16, 17, 17, 17, 18, 18, 18, 18, 19, 20, 21, 21, 21, 22, 23, 23, 23, 24, 25, 26], dtype=np.int32)
_EDGE_DST = np.array([1, 0, 2, 4, 6, 1, 3, 7, 2, 4, 24, 1, 3, 5, 4, 6, 17, 19, 1, 5, 7, 8, 10, 2, 6, 23, 6, 9, 16, 8, 10, 13, 6, 9, 11, 10, 12, 23, 11, 13, 21, 9, 12, 14, 13, 15, 20, 14, 16, 18, 8, 15, 17, 5, 16, 18, 15, 17, 25, 26, 5, 14, 12, 22, 23, 21, 7, 11, 21, 3, 18, 18], dtype=np.int32)
_N = 72
_B, _C, _F = 32, 128, 4096
_NCHUNK, _CW = 10, 71
_NE = 27
_NP = 32
_CD = _NE - 1
_PPAD = 272
_GCH = (128, 128, 16)

_A_np = np.zeros((_NP, _NP), dtype=np.float32)
for _s, _d in zip(_EDGE_SRC, _EDGE_DST):
    _A_np[_d, _s] += 1.0
_A0_np = np.zeros((_NP, _NP), dtype=np.float32)
_A0_np[:, :_CD] = _A_np[:, 1:_NE]
_A0_np[:, _NP - 1] = _A_np[:, 0]
_A2_np = np.zeros((_NP, _NP), dtype=np.float32)
_A2_np[:_CD, :] = _A_np[1:_NE, :]
_A2_np[_NP - 1, :] = _A_np[0, :]
_S_np = np.zeros((_B, _B * _NP), dtype=np.float32)
for _i in range(_B):
    _S_np[_i, _i * _NP + _NP - 1] = 1.0

_rng = np.random.default_rng(0)
_II = np.stack([_rng.choice(_F, _NCHUNK * _CW, replace=False)
                for _ in range(_B)])
_IDX_np = np.zeros((_B, _PPAD), dtype=np.int32)
for _i in range(_B):
    for _j in range(_NCHUNK):
        _IDX_np[_i, _j * _CD:(_j + 1) * _CD] = (
            _II[_i, _j * _CW:_j * _CW + _CD].astype(np.int32) + _i * _F)
    _IDX_np[_i, _NCHUNK * _CD:] = _i * _F

_MMARGS = dict(dimension_numbers=(((1,), (0,)), ((), ())),
               preferred_element_type=jnp.float32)


def _mm(a, b):
    return lax.dot_general(a, b, **_MMARGS)


def _mmh(a, b):
    return lax.dot_general(a, b, precision=lax.Precision.HIGHEST, **_MMARGS)


def _k1_body(x_ref, xt_ref):
    xt_ref[0] = x_ref[0].T


def _transpose_call(x):
    return pl.pallas_call(
        _k1_body,
        grid=(_B,),
        in_specs=[pl.BlockSpec((1, _C, _F), lambda s: (s, 0, 0))],
        out_specs=pl.BlockSpec((1, _F, _C), lambda s: (s, 0, 0)),
        out_shape=jax.ShapeDtypeStruct((_B, _F, _C), jnp.float32),
    )(x)


def _k2_body(xt_hbm, idx_hbm, xg_hbm, idx_v, rows_v, sem):
    wid = lax.axis_index("s") * 2 + lax.axis_index("c")
    pltpu.sync_copy(idx_hbm.at[wid], idx_v)
    off = 0
    copies = []
    for g in _GCH:
        copies.append(pltpu.async_copy(
            xt_hbm.at[idx_v.at[pl.ds(off, g)]],
            rows_v.at[pl.ds(off, g)], sem))
        off += g
    for c in copies:
        c.wait()
    outs = []
    for j in range(_NCHUNK):
        outs.append(pltpu.async_copy(
            rows_v.at[pl.ds(j * _CD, _NP)],
            xg_hbm.at[j, pl.ds(wid * _NP, _NP)], sem))
    for c in outs:
        c.wait()


def _gather_call(xt, idx):
    mesh = plsc.VectorSubcoreMesh(core_axis_name="c", subcore_axis_name="s")
    k = pl.kernel(
        _k2_body,
        out_type=jax.ShapeDtypeStruct((_NCHUNK, _B * _NP, _C), jnp.float32),
        mesh=mesh,
        scratch_types=[
            pltpu.VMEM((_PPAD,), jnp.int32),
            pltpu.VMEM((_PPAD, _C), jnp.float32),
            pltpu.SemaphoreType.DMA,
        ],
    )
    return k(xt.reshape(_B * _F, _C), idx)


def _blockagg(M, h):
    return jnp.concatenate(
        [_mmh(M, h[s * _NP:(s + 1) * _NP]) for s in range(_B)], axis=0)


def _k3_body(xg_ref, A_ref, A0_ref, A2_ref, S_ref, W0_ref, b0_ref, W1_ref,
             b1_ref, W2_ref, b2_ref, Wc_ref, bc_ref, out_ref, X3_ref):
    j = pl.program_id(0)

    @pl.when(j == 0)
    def _():
        X3_ref[...] = jnp.zeros((_B * _NP, _C), jnp.float32)

    maskfb = lax.rem(
        lax.broadcasted_iota(jnp.int32, (_B * _NP, 1), 0), _NP) == (_NP - 1)
    Xin = jnp.where(maskfb, X3_ref[...], xg_ref[0])
    X = jnp.maximum(_blockagg(A0_ref[...], _mm(Xin, W0_ref[...])) + b0_ref[...], 0.0)
    X = jnp.maximum(_blockagg(A_ref[...], _mm(X, W1_ref[...])) + b1_ref[...], 0.0)
    X = jnp.maximum(_blockagg(A2_ref[...], _mm(X, W2_ref[...])) + b2_ref[...], 0.0)
    X3_ref[...] = X

    @pl.when(j == _NCHUNK - 1)
    def _():
        fb = _mmh(S_ref[...], X)
        out_ref[...] = _mm(fb, Wc_ref[...]) + bc_ref[...]


def _gcn_call(xg, A, A0, A2, S, W0, b0, W1, b1, W2, b2, Wc, bc):
    full = lambda *shape: pl.BlockSpec(shape, lambda j: (0,) * len(shape))
    return pl.pallas_call(
        _k3_body,
        grid=(_NCHUNK,),
        in_specs=[
            pl.BlockSpec((1, _B * _NP, _C), lambda j: (j, 0, 0)),
            full(_NP, _NP), full(_NP, _NP), full(_NP, _NP),
            full(_B, _B * _NP),
            full(_C, 256), full(1, 256),
            full(256, 256), full(1, 256),
            full(256, _C), full(1, _C),
            full(_C, 1), full(1, 1),
        ],
        out_specs=pl.BlockSpec((_B, 1), lambda j: (0, 0)),
        out_shape=jax.ShapeDtypeStruct((_B, 1), jnp.float32),
        scratch_shapes=[pltpu.VMEM((_B * _NP, _C), jnp.float32)],
    )(xg, A, A0, A2, S, W0, b0, W1, b1, W2, b2, Wc, bc)


def kernel(x, W0, b0, W1, b1, W2, b2, Wc, bc):
    xt = _transpose_call(x)
    xg = _gather_call(xt, jnp.asarray(_IDX_np))
    return _gcn_call(
        xg, jnp.asarray(_A_np), jnp.asarray(_A0_np), jnp.asarray(_A2_np),
        jnp.asarray(_S_np),
        W0, b0.reshape(1, -1), W1, b1.reshape(1, -1), W2, b2.reshape(1, -1),
        Wc, bc.reshape(1, 1))


# speedup vs baseline: 2.3117x; 1.1280x over previous
import functools

import jax
import jax.numpy as jnp
import numpy as np
from jax import lax
from jax.experimental import pallas as pl
from jax.experimental.pallas import tpu as pltpu
from jax.experimental.pallas import tpu_sc as plsc

_EDGE_SRC = np.array([0, 1, 1, 1, 1, 2, 2, 2, 3, 3, 3, 4, 4, 4, 5, 5, 5, 5, 6, 6, 6, 6, 6, 7, 7, 7, 8, 8, 8, 9, 9, 9, 10, 10, 10, 11, 11, 11, 12, 12, 12, 13, 13, 13, 14, 14, 14, 15, 15, 15, 16, 16, 16, 17, 17, 17, 18, 18, 18, 18, 19, 20, 21, 21, 21, 22, 23, 23, 23, 24, 25, 26], dtype=np.int32)
_EDGE_DST = np.array([1, 0, 2, 4, 6, 1, 3, 7, 2, 4, 24, 1, 3, 5, 4, 6, 17, 19, 1, 5, 7, 8, 10, 2, 6, 23, 6, 9, 16, 8, 10, 13, 6, 9, 11, 10, 12, 23, 11, 13, 21, 9, 12, 14, 13, 15, 20, 14, 16, 18, 8, 15, 17, 5, 16, 18, 15, 17, 25, 26, 5, 14, 12, 22, 23, 21, 7, 11, 21, 3, 18, 18], dtype=np.int32)
_N = 72
_B, _C, _F = 32, 128, 4096
_NCHUNK, _CW = 10, 71
_NE = 27
_NP = 32
_CD = _NE - 1
_PPAD = 272
_GCH = (128, 128, 16)

_A_np = np.zeros((_NP, _NP), dtype=np.float32)
for _s, _d in zip(_EDGE_SRC, _EDGE_DST):
    _A_np[_d, _s] += 1.0
_A0_np = np.zeros((_NP, _NP), dtype=np.float32)
_A0_np[:, :_CD] = _A_np[:, 1:_NE]
_A0_np[:, _NP - 1] = _A_np[:, 0]
_A2_np = np.zeros((_NP, _NP), dtype=np.float32)
_A2_np[:_CD, :] = _A_np[1:_NE, :]
_A2_np[_NP - 1, :] = _A_np[0, :]
_S_np = np.zeros((_B, _B * _NP), dtype=np.float32)
for _i in range(_B):
    _S_np[_i, _i * _NP + _NP - 1] = 1.0

_rng = np.random.default_rng(0)
_II = np.stack([_rng.choice(_F, _NCHUNK * _CW, replace=False)
                for _ in range(_B)])
_IDX_np = np.zeros((_B, _PPAD), dtype=np.int32)
for _i in range(_B):
    for _j in range(_NCHUNK):
        _IDX_np[_i, _j * _CD:(_j + 1) * _CD] = (
            _II[_i, _j * _CW:_j * _CW + _CD].astype(np.int32) + _i * _F)
    _IDX_np[_i, _NCHUNK * _CD:] = _i * _F

_MMARGS = dict(dimension_numbers=(((1,), (0,)), ((), ())),
               preferred_element_type=jnp.float32)


def _mm(a, b):
    return lax.dot_general(a, b, **_MMARGS)


def _mmh(a, b):
    return lax.dot_general(a, b, precision=lax.Precision.HIGHEST, **_MMARGS)


def _k1_body(x_ref, xt_ref):
    xt_ref[0] = x_ref[0].T


def _transpose_call(x):
    return pl.pallas_call(
        _k1_body,
        grid=(_B,),
        in_specs=[pl.BlockSpec((1, _C, _F), lambda s: (s, 0, 0))],
        out_specs=pl.BlockSpec((1, _F, _C), lambda s: (s, 0, 0)),
        out_shape=jax.ShapeDtypeStruct((_B, _F, _C), jnp.float32),
    )(x)


def _k2_body(xt_hbm, idx_hbm, xg_hbm, idx_v, rows_v, sem):
    wid = lax.axis_index("s") * 2 + lax.axis_index("c")
    pltpu.sync_copy(idx_hbm.at[wid], idx_v)
    off = 0
    copies = []
    for g in _GCH:
        copies.append(pltpu.async_copy(
            xt_hbm.at[idx_v.at[pl.ds(off, g)]],
            rows_v.at[pl.ds(off, g)], sem))
        off += g
    for c in copies:
        c.wait()
    outs = []
    for j in range(_NCHUNK):
        outs.append(pltpu.async_copy(
            rows_v.at[pl.ds(j * _CD, _NP)],
            xg_hbm.at[j, pl.ds(wid * _NP, _NP)], sem))
    for c in outs:
        c.wait()


def _gather_call(xt, idx):
    mesh = plsc.VectorSubcoreMesh(core_axis_name="c", subcore_axis_name="s")
    k = pl.kernel(
        _k2_body,
        out_type=jax.ShapeDtypeStruct((_NCHUNK, _B * _NP, _C), jnp.float32),
        mesh=mesh,
        scratch_types=[
            pltpu.VMEM((_PPAD,), jnp.int32),
            pltpu.VMEM((_PPAD, _C), jnp.float32),
            pltpu.SemaphoreType.DMA,
        ],
    )
    return k(xt.reshape(_B * _F, _C), idx)


def _blockagg(M, h):
    h1 = h.astype(jnp.bfloat16).astype(jnp.float32)
    h2 = h - h1
    return jnp.concatenate(
        [_mm(M, h1[s * _NP:(s + 1) * _NP]) + _mm(M, h2[s * _NP:(s + 1) * _NP])
         for s in range(_B)], axis=0)


def _k3_body(xg_ref, A_ref, A0_ref, A2_ref, S_ref, W0_ref, b0_ref, W1_ref,
             b1_ref, W2_ref, b2_ref, Wc_ref, bc_ref, out_ref, X3_ref):
    j = pl.program_id(0)

    @pl.when(j == 0)
    def _():
        X3_ref[...] = jnp.zeros((_B * _NP, _C), jnp.float32)

    maskfb = lax.rem(
        lax.broadcasted_iota(jnp.int32, (_B * _NP, 1), 0), _NP) == (_NP - 1)
    Xin = jnp.where(maskfb, X3_ref[...], xg_ref[0])
    X = jnp.maximum(_blockagg(A0_ref[...], _mm(Xin, W0_ref[...])) + b0_ref[...], 0.0)
    X = jnp.maximum(_blockagg(A_ref[...], _mm(X, W1_ref[...])) + b1_ref[...], 0.0)
    X = jnp.maximum(_blockagg(A2_ref[...], _mm(X, W2_ref[...])) + b2_ref[...], 0.0)
    X3_ref[...] = X

    @pl.when(j == _NCHUNK - 1)
    def _():
        fb = _mmh(S_ref[...], X)
        out_ref[...] = _mm(fb, Wc_ref[...]) + bc_ref[...]


def _gcn_call(xg, A, A0, A2, S, W0, b0, W1, b1, W2, b2, Wc, bc):
    full = lambda *shape: pl.BlockSpec(shape, lambda j: (0,) * len(shape))
    return pl.pallas_call(
        _k3_body,
        grid=(_NCHUNK,),
        in_specs=[
            pl.BlockSpec((1, _B * _NP, _C), lambda j: (j, 0, 0)),
            full(_NP, _NP), full(_NP, _NP), full(_NP, _NP),
            full(_B, _B * _NP),
            full(_C, 256), full(1, 256),
            full(256, 256), full(1, 256),
            full(256, _C), full(1, _C),
            full(_C, 1), full(1, 1),
        ],
        out_specs=pl.BlockSpec((_B, 1), lambda j: (0, 0)),
        out_shape=jax.ShapeDtypeStruct((_B, 1), jnp.float32),
        scratch_shapes=[pltpu.VMEM((_B * _NP, _C), jnp.float32)],
    )(xg, A, A0, A2, S, W0, b0, W1, b1, W2, b2, Wc, bc)


def kernel(x, W0, b0, W1, b1, W2, b2, Wc, bc):
    xt = _transpose_call(x)
    xg = _gather_call(xt, jnp.asarray(_IDX_np))
    return _gcn_call(
        xg, jnp.asarray(_A_np), jnp.asarray(_A0_np), jnp.asarray(_A2_np),
        jnp.asarray(_S_np),
        W0, b0.reshape(1, -1), W1, b1.reshape(1, -1), W2, b2.reshape(1, -1),
        Wc, bc.reshape(1, 1))
